# baseline (speedup 1.0000x reference)
import functools

import jax
import jax.numpy as jnp
from jax import lax
from jax.experimental import pallas as pl
from jax.experimental.pallas import tpu as pltpu
from jax.experimental.pallas import tpu_sc as plsc

_B = 16384
_D = 512
_OUT_LO = 120
_OUT_HI = 136
_NC = 2
_NS = 16
_NW = _NC * _NS
_G = _B * _D // 1024
_GPW = _G // _NW
_CG = 32
_NCHUNK = _GPW // _CG
_NBUF = 3


def _overwrite(buf, lane, two):
    for j in range(_CG // 16):
        g16 = lane + (j * 16)
        for s, l in ((0, 120), (4, 120), (1, 8), (5, 8)):
            plsc.store_scatter(
                buf,
                [g16, jnp.full((16,), s, jnp.int32),
                 jnp.full((16,), l, jnp.int32)],
                two)


def _sc_body(x_hbm, out_hbm, *scratch):
    bufs = scratch[:_NBUF]
    sin = scratch[_NBUF:2 * _NBUF]
    sout = scratch[2 * _NBUF:3 * _NBUF]
    wid = lax.axis_index("s") * _NC + lax.axis_index("c")
    base = wid * _GPW
    lane = lax.iota(jnp.int32, 16)
    two = jnp.full((16,), 2.0, jnp.float32)
    in_cp = [None] * _NBUF
    out_cp = [None] * _NBUF
    for i in range(min(_NBUF, _NCHUNK)):
        in_cp[i] = pltpu.async_copy(
            x_hbm.at[pl.ds(base + i * _CG, _CG)], bufs[i], sin[i])
    for i in range(_NCHUNK):
        b = i % _NBUF
        in_cp[b].wait()
        _overwrite(bufs[b], lane, two)
        out_cp[b] = pltpu.async_copy(
            bufs[b], out_hbm.at[pl.ds(base + i * _CG, _CG)], sout[b])
        nxt = i + _NBUF
        if nxt < _NCHUNK:
            out_cp[b].wait()
            in_cp[b] = pltpu.async_copy(
                x_hbm.at[pl.ds(base + nxt * _CG, _CG)], bufs[b], sin[b])
    for i in range(max(0, _NCHUNK - _NBUF), _NCHUNK):
        out_cp[i % _NBUF].wait()


_sc_call = functools.partial(
    pl.kernel,
    out_type=jax.ShapeDtypeStruct((_G, 8, 128), jnp.float32),
    mesh=plsc.VectorSubcoreMesh(core_axis_name="c", subcore_axis_name="s"),
    scratch_types=(
        [pltpu.VMEM((_CG, 8, 128), jnp.float32)] * _NBUF
        + [pltpu.SemaphoreType.DMA] * (2 * _NBUF)
    ),
    compiler_params=pltpu.CompilerParams(
        needs_layout_passes=False, use_tc_tiling_on_sc=True),
)(_sc_body)


def kernel(x_bd):
    out = _sc_call(x_bd.reshape(_G, 8, 128))
    return out.reshape(_B, 1, _D)
